# Initial kernel scaffold; baseline (speedup 1.0000x reference)
import functools

import jax
import jax.numpy as jnp
import numpy as np
from jax import lax
from jax.experimental import pallas as pl
from jax.experimental.pallas import tpu as pltpu
from jax.experimental.pallas import tpu_sc as plsc

_FIELD_DIMS = [100000] * 26
_F = len(_FIELD_DIMS)
_B = 16384
_NW = 32
_BPW = _B // _NW
_IPW = _BPW * _F
_NCH = _IPW // 16
_PERIOD = 208


def _offs_pattern() -> np.ndarray:
    offs = np.array((0, *np.cumsum(_FIELD_DIMS)[:-1]), dtype=np.int32)
    return np.array([offs[i % _F] for i in range(_PERIOD)], dtype=np.int32)


_OFFS = _offs_pattern()


def _sc_kernel(x_hbm, t_hbm, offs_hbm, bias_hbm, out_hbm,
               idx_v, val_v, offs_v, bias_v, o_v, sem):
    info = plsc.get_sparse_core_info()
    nc = info.num_cores
    wid = lax.axis_index("s") * nc + lax.axis_index("c")
    base = wid * _BPW

    pltpu.sync_copy(x_hbm.at[pl.ds(base * _F, _IPW)], idx_v)
    pltpu.sync_copy(offs_hbm, offs_v)
    pltpu.sync_copy(bias_hbm, bias_v)

    def add_off(c, carry):
        off = offs_v[pl.ds((c % 13) * 16, 16)]
        idx_v[pl.ds(c * 16, 16)] = idx_v[pl.ds(c * 16, 16)] + off
        return carry

    lax.fori_loop(0, _NCH, add_off, 0)

    pltpu.async_copy(t_hbm.at[idx_v], val_v, sem).wait()

    lane26 = lax.iota(jnp.int32, 16) * _F
    biasv = bias_v[...]

    def reduce_chunk(cb, carry):
        b0 = cb * (16 * _F)
        acc = biasv
        for j in range(_F):
            acc = acc + plsc.load_gather(val_v, [lane26 + (b0 + j)])
        o_v[pl.ds(cb * 16, 16)] = acc
        return carry

    lax.fori_loop(0, _BPW // 16, reduce_chunk, 0)

    pltpu.sync_copy(o_v, out_hbm.at[pl.ds(base, _BPW)])


def kernel(x, table, bias):
    x_flat = x.reshape(-1).astype(jnp.int32)
    table_flat = table.reshape(-1)
    offs = jnp.asarray(_OFFS)
    bias16 = jnp.broadcast_to(bias.astype(jnp.float32), (16,))

    mesh = plsc.VectorSubcoreMesh(core_axis_name="c", subcore_axis_name="s")
    run = functools.partial(
        pl.kernel,
        out_type=jax.ShapeDtypeStruct((_B,), jnp.float32),
        mesh=mesh,
        scratch_types=[
            pltpu.VMEM((_IPW,), jnp.int32),
            pltpu.VMEM((_IPW,), jnp.float32),
            pltpu.VMEM((_PERIOD,), jnp.int32),
            pltpu.VMEM((16,), jnp.float32),
            pltpu.VMEM((_BPW,), jnp.float32),
            pltpu.SemaphoreType.DMA,
        ],
    )(_sc_kernel)
    out = run(x_flat, table_flat, offs, bias16)
    return out.reshape(_B, 1)


# speedup vs baseline: 1.2379x; 1.2379x over previous
import functools

import jax
import jax.numpy as jnp
from jax import lax
from jax.experimental import pallas as pl
from jax.experimental.pallas import tpu as pltpu
from jax.experimental.pallas import tpu_sc as plsc

_F = 26
_FDIM = 100000
_B = 16384
_NW = 32
_BPW = _B // _NW
_IPW = _BPW * _F
_NCH = _IPW // 16


def _sc_kernel(x_hbm, t_hbm, bias_hbm, out_hbm, idx_v, val_v, bias_v, o_v, sem):
    info = plsc.get_sparse_core_info()
    nc = info.num_cores
    wid = lax.axis_index("s") * nc + lax.axis_index("c")
    base = wid * _BPW

    copies = [
        pltpu.async_copy(
            x_hbm.at[pl.ds(f * _B + base, _BPW)],
            idx_v.at[pl.ds(f * _BPW, _BPW)],
            sem,
        )
        for f in range(_F)
    ]
    pltpu.sync_copy(bias_hbm, bias_v)
    for c in copies:
        c.wait()

    def add_off(c, carry):
        off = (c // (_BPW // 16)) * _FDIM
        idx_v[pl.ds(c * 16, 16)] = idx_v[pl.ds(c * 16, 16)] + off
        return carry

    lax.fori_loop(0, _NCH, add_off, 0)

    pltpu.async_copy(t_hbm.at[idx_v], val_v, sem).wait()

    biasv = bias_v[...]

    def reduce_chunk(cb, carry):
        b0 = cb * 16
        acc = biasv
        for f in range(_F):
            acc = acc + val_v[pl.ds(f * _BPW + b0, 16)]
        o_v[pl.ds(b0, 16)] = acc
        return carry

    lax.fori_loop(0, _BPW // 16, reduce_chunk, 0)

    pltpu.sync_copy(o_v, out_hbm.at[pl.ds(base, _BPW)])


def kernel(x, table, bias):
    xt_flat = x.astype(jnp.int32).T.reshape(-1)
    table_flat = table.reshape(-1)
    bias16 = jnp.broadcast_to(bias.astype(jnp.float32), (16,))

    mesh = plsc.VectorSubcoreMesh(core_axis_name="c", subcore_axis_name="s")
    run = functools.partial(
        pl.kernel,
        out_type=jax.ShapeDtypeStruct((_B,), jnp.float32),
        mesh=mesh,
        scratch_types=[
            pltpu.VMEM((_IPW,), jnp.int32),
            pltpu.VMEM((_IPW,), jnp.float32),
            pltpu.VMEM((16,), jnp.float32),
            pltpu.VMEM((_BPW,), jnp.float32),
            pltpu.SemaphoreType.DMA,
        ],
    )(_sc_kernel)
    out = run(xt_flat, table_flat, bias16)
    return out.reshape(_B, 1)
